# baseline (speedup 1.0000x reference)
import functools

import jax
import jax.numpy as jnp
from jax import lax
from jax.experimental import pallas as pl
from jax.experimental.pallas import tpu as pltpu
from jax.experimental.pallas import tpu_sc as plsc

B = 16384
L = 20
D = 32
NC = 2
NS = 16
NW = NC * NS
BPW = B // NW
GB = 4
G = GB * L
NSTEP = BPW // GB
NBUF = 4
SCALE = 1.0 / L


def _make_kernel():
    mesh = plsc.VectorSubcoreMesh(core_axis_name="c", subcore_axis_name="s")

    @functools.partial(
        pl.kernel,
        mesh=mesh,
        out_type=jax.ShapeDtypeStruct((B, D), jnp.float32),
        scratch_types=[
            pltpu.VMEM((NSTEP, G), jnp.int32),
            pltpu.VMEM((NBUF, G, 128), jnp.float32),
            pltpu.VMEM((BPW, D), jnp.float32),
            pltpu.SemaphoreType.DMA((NBUF,)),
        ],
        compiler_params=pltpu.CompilerParams(use_tc_tiling_on_sc=False),
    )
    def k(idx_hbm, table_hbm, out_hbm, idx_v, rows_v, out_v, sems):
        wid = lax.axis_index("s") * NC + lax.axis_index("c")

        pltpu.sync_copy(idx_hbm.at[wid], idx_v)

        def fire(s, b):
            pltpu.async_copy(
                table_hbm.at[idx_v.at[s]], rows_v.at[b], sems.at[b]
            )

        def wait(b):
            pltpu.make_async_copy(
                table_hbm.at[pl.ds(0, G)], rows_v.at[b], sems.at[b]
            ).wait()

        def compute(s, b):
            for bb in range(GB):
                r0 = bb * L
                acc0 = rows_v[b, r0, 0:16]
                acc1 = rows_v[b, r0, 16:32]
                for l in range(1, L):
                    acc0 = acc0 + rows_v[b, r0 + l, 0:16]
                    acc1 = acc1 + rows_v[b, r0 + l, 16:32]
                row = s * GB + bb
                out_v[row, 0:16] = acc0 * SCALE
                out_v[row, 16:32] = acc1 * SCALE

        for b in range(NBUF):
            fire(b, b)

        def body(i, carry):
            s0 = i * NBUF
            for b in range(NBUF):
                s = s0 + b
                wait(b)
                compute(s, b)
                fire(s + NBUF, b)
            return carry

        lax.fori_loop(0, (NSTEP - NBUF) // NBUF, body, 0)

        for b in range(NBUF):
            wait(b)
            compute(NSTEP - NBUF + b, b)

        pltpu.sync_copy(out_v, out_hbm.at[pl.ds(wid * BPW, BPW)])

    return k


_sc_kernel = _make_kernel()


@jax.jit
def kernel(context_idxs, input_emb):
    idx = context_idxs.astype(jnp.int32).reshape(NW, NSTEP, G) // 4
    table = input_emb.reshape(250000, 128)
    return _sc_kernel(idx, table)


# speedup vs baseline: 1.0715x; 1.0715x over previous
import functools

import jax
import jax.numpy as jnp
from jax import lax
from jax.experimental import pallas as pl
from jax.experimental.pallas import tpu as pltpu
from jax.experimental.pallas import tpu_sc as plsc

B = 16384
L = 20
D = 32
NC = 2
NS = 16
NW = NC * NS
BPW = B // NW
GB = 4
G = GB * L
NSTEP = BPW // GB
NBUF = 4
SCALE = 1.0 / L


def _make_kernel():
    mesh = plsc.VectorSubcoreMesh(core_axis_name="c", subcore_axis_name="s")

    @functools.partial(
        pl.kernel,
        mesh=mesh,
        out_type=jax.ShapeDtypeStruct((B, D), jnp.float32),
        scratch_types=[
            pltpu.VMEM((NSTEP, G), jnp.int32),
            pltpu.VMEM((NBUF, G, D), jnp.float32),
            pltpu.VMEM((BPW, D), jnp.float32),
            pltpu.SemaphoreType.DMA((NBUF,)),
        ],
        compiler_params=pltpu.CompilerParams(use_tc_tiling_on_sc=False),
    )
    def k(idx_hbm, table_hbm, out_hbm, idx_v, rows_v, out_v, sems):
        wid = lax.axis_index("s") * NC + lax.axis_index("c")

        pltpu.sync_copy(idx_hbm.at[wid], idx_v)

        def fire(s, b):
            for j in range(G // 16):
                iv = idx_v[s, pl.ds(j * 16, 16)]
                pltpu.async_copy(
                    table_hbm.at[iv],
                    rows_v.at[b].at[pl.ds(j * 16, 16)],
                    sems.at[b],
                )

        def wait(b):
            pltpu.make_async_copy(
                table_hbm.at[pl.ds(0, G)], rows_v.at[b], sems.at[b]
            ).wait()

        def compute(s, b):
            for bb in range(GB):
                r0 = bb * L
                acc0 = rows_v[b, r0, 0:16]
                acc1 = rows_v[b, r0, 16:32]
                for l in range(1, L):
                    acc0 = acc0 + rows_v[b, r0 + l, 0:16]
                    acc1 = acc1 + rows_v[b, r0 + l, 16:32]
                row = s * GB + bb
                out_v[row, 0:16] = acc0 * SCALE
                out_v[row, 16:32] = acc1 * SCALE

        for b in range(NBUF):
            fire(b, b)

        def body(i, carry):
            s0 = i * NBUF
            for b in range(NBUF):
                s = s0 + b
                wait(b)
                compute(s, b)
                fire(s + NBUF, b)
            return carry

        lax.fori_loop(0, (NSTEP - NBUF) // NBUF, body, 0)

        for b in range(NBUF):
            wait(b)
            compute(NSTEP - NBUF + b, b)

        pltpu.sync_copy(out_v, out_hbm.at[pl.ds(wid * BPW, BPW)])

    return k


_sc_kernel = _make_kernel()


@jax.jit
def kernel(context_idxs, input_emb):
    idx = context_idxs.astype(jnp.int32).reshape(NW, NSTEP, G)
    return _sc_kernel(idx, input_emb)


# speedup vs baseline: 1.0802x; 1.0081x over previous
import functools

import jax
import jax.numpy as jnp
from jax import lax
from jax.experimental import pallas as pl
from jax.experimental.pallas import tpu as pltpu
from jax.experimental.pallas import tpu_sc as plsc

B = 16384
L = 20
D = 32
NC = 2
NS = 16
NW = NC * NS
BPW = B // NW
GB = 4
G = GB * L
NSTEP = BPW // GB
NBUF = 4
SCALE = 1.0 / L


def _make_kernel():
    mesh = plsc.VectorSubcoreMesh(core_axis_name="c", subcore_axis_name="s")

    @functools.partial(
        pl.kernel,
        mesh=mesh,
        out_type=jax.ShapeDtypeStruct((B, D), jnp.float32),
        scratch_types=[
            pltpu.VMEM((NSTEP, G), jnp.int32),
            pltpu.VMEM((NBUF, G, D), jnp.float32),
            pltpu.VMEM((BPW, D), jnp.float32),
            pltpu.SemaphoreType.DMA((NBUF,)),
        ],
        compiler_params=pltpu.CompilerParams(use_tc_tiling_on_sc=False),
    )
    def k(idx_hbm, table_hbm, out_hbm, idx_v, rows_v, out_v, sems):
        wid = lax.axis_index("s") * NC + lax.axis_index("c")

        pltpu.sync_copy(idx_hbm.at[wid], idx_v)

        def fire(s, b):
            pltpu.async_copy(
                table_hbm.at[idx_v.at[s]], rows_v.at[b], sems.at[b]
            )

        def wait(b):
            pltpu.make_async_copy(
                table_hbm.at[pl.ds(0, G)], rows_v.at[b], sems.at[b]
            ).wait()

        def compute(s, b):
            for bb in range(GB):
                r0 = bb * L
                acc0 = rows_v[b, r0, 0:16]
                acc1 = rows_v[b, r0, 16:32]
                for l in range(1, L):
                    acc0 = acc0 + rows_v[b, r0 + l, 0:16]
                    acc1 = acc1 + rows_v[b, r0 + l, 16:32]
                row = s * GB + bb
                out_v[row, 0:16] = acc0 * SCALE
                out_v[row, 16:32] = acc1 * SCALE

        for b in range(NBUF):
            fire(b, b)

        def body(i, carry):
            s0 = i * NBUF
            for b in range(NBUF):
                s = s0 + b
                wait(b)
                compute(s, b)
                fire(s + NBUF, b)
            return carry

        lax.fori_loop(0, (NSTEP - NBUF) // NBUF, body, 0)

        for b in range(NBUF):
            wait(b)
            compute(NSTEP - NBUF + b, b)

        pltpu.sync_copy(out_v, out_hbm.at[pl.ds(wid * BPW, BPW)])

    return k


_sc_kernel = _make_kernel()


@jax.jit
def kernel(context_idxs, input_emb):
    idx = context_idxs.astype(jnp.int32).reshape(NW, NSTEP, G)
    return _sc_kernel(idx, input_emb)
